# baseline (speedup 1.0000x reference)
import functools

import jax
import jax.numpy as jnp
from jax import lax
from jax.experimental import pallas as pl
from jax.experimental.pallas import tpu as pltpu
from jax.experimental.pallas import tpu_sc as plsc

_SC_HEAD = 768
_K = 128
_TB = 256


def _tc_body(y_ref, x_ref, loc_ref, W2_ref, bf_ref, W1_ref, w0_ref, c_ref,
             d_ref, ovt_ref):
    xb = x_ref[0]
    xb = jnp.where(jnp.isnan(xb), 0.0, xb)
    xa = jnp.dot(xb, W2_ref[...], preferred_element_type=jnp.float32)
    xa = xa + bf_ref[...]
    k = jax.lax.broadcasted_iota(jnp.int32, xa.shape, 1) % 6
    feat = jnp.where(k == 0, xa, jnp.sin(xa))
    tp = jnp.dot(feat, W1_ref[...], preferred_element_type=jnp.float32)
    base = loc_ref[...] + tp + c_ref[...]
    yb = y_ref[0]
    nanm = jnp.isnan(yb)
    y0 = jnp.where(nanm, 0.0, yb)
    nf = nanm.astype(jnp.float32)
    w0 = w0_ref[...]
    dl = d_ref[...]
    for i in range(8):
        ovt_ref[0, i] = base + y0[:, i:i + 1] * w0 + nf[:, i:i + 1] * dl


def _val_time_tc(y, x, local_emb_table, W2, b_flat, vt_W1, w0row, crow, drow):
    bs, L, d_y = y.shape
    d_x = x.shape[-1]
    d_model = local_emb_table.shape[-1]
    TD = W2.shape[1]
    nt = L // _TB
    vt4 = pl.pallas_call(
        _tc_body,
        grid=(bs, nt),
        in_specs=[
            pl.BlockSpec((1, _TB, d_y), lambda b, t: (b, t, 0)),
            pl.BlockSpec((1, _TB, d_x), lambda b, t: (b, t, 0)),
            pl.BlockSpec((_TB, d_model), lambda b, t: (t, 0)),
            pl.BlockSpec((d_x, TD), lambda b, t: (0, 0)),
            pl.BlockSpec((1, TD), lambda b, t: (0, 0)),
            pl.BlockSpec((TD, d_model), lambda b, t: (0, 0)),
            pl.BlockSpec((1, d_model), lambda b, t: (0, 0)),
            pl.BlockSpec((1, d_model), lambda b, t: (0, 0)),
            pl.BlockSpec((1, d_model), lambda b, t: (0, 0)),
        ],
        out_specs=pl.BlockSpec((1, d_y, _TB, d_model),
                               lambda b, t: (b, 0, t, 0)),
        out_shape=jax.ShapeDtypeStruct((bs, d_y, L, d_model), jnp.float32),
        compiler_params=pltpu.CompilerParams(
            dimension_semantics=("parallel", "parallel")),
    )(y, x, local_emb_table, W2, b_flat, vt_W1, w0row, crow, drow)
    return vt4.reshape(bs, d_y * L, d_model)


def _space_sc_head(space_table, bs, L, d_model):
    d_y = space_table.shape[0]
    info = plsc.get_sparse_core_info()
    NW = info.num_cores * info.num_subcores
    mesh = plsc.VectorSubcoreMesh(core_axis_name="c", subcore_axis_name="s")

    @functools.partial(
        pl.kernel, mesh=mesh,
        out_type=jax.ShapeDtypeStruct((NW, L, d_model), jnp.float32),
        scratch_types=[
            pltpu.VMEM((_K,), jnp.int32),
            pltpu.VMEM((_K, d_model), jnp.float32),
            pltpu.SemaphoreType.DMA,
        ],
    )
    def k(table_hbm, out_hbm, idx_v, buf, sem):
        wid = lax.axis_index("s") * info.num_cores + lax.axis_index("c")
        tid = wid % d_y
        splat = jnp.full((16,), tid, dtype=jnp.int32)
        for c in range(_K // 16):
            idx_v[pl.ds(c * 16, 16)] = splat
        pltpu.async_copy(table_hbm.at[idx_v], buf, sem).wait()
        copies = [
            pltpu.async_copy(buf, out_hbm.at[wid, pl.ds(j * _K, _K)], sem)
            for j in range(_SC_HEAD // _K)
        ]
        for c in copies:
            c.wait()

    return k(space_table)


def _space_tc_tail(space_table, sp_head, bs, L, d_model):
    d_y = space_table.shape[0]
    t0 = _SC_HEAD // _TB
    n_tail = (L - _SC_HEAD) // _TB

    def body(sp_ref, alias_ref, out_ref):
        del alias_ref
        for i in range(d_y):
            out_ref[0, i] = jnp.broadcast_to(sp_ref[i:i + 1, :],
                                             (_TB, d_model))

    sp4 = pl.pallas_call(
        body,
        grid=(bs, n_tail),
        in_specs=[
            pl.BlockSpec((d_y, d_model), lambda b, t: (0, 0)),
            pl.BlockSpec(memory_space=pltpu.MemorySpace.HBM),
        ],
        out_specs=pl.BlockSpec((1, d_y, _TB, d_model),
                               lambda b, t: (b, 0, t + t0, 0)),
        out_shape=jax.ShapeDtypeStruct((bs, d_y, L, d_model), jnp.float32),
        input_output_aliases={1: 0},
        compiler_params=pltpu.CompilerParams(
            dimension_semantics=("parallel", "parallel")),
    )(space_table, sp_head)
    return sp4.reshape(bs, d_y * L, d_model)


def kernel(y, x, local_emb_table, time_w, time_b, vt_W, vt_b, space_table,
           given_table):
    bs, L, d_y = y.shape
    d_x = x.shape[-1]
    d_model = local_emb_table.shape[-1]
    E = time_w.shape[1]
    TD = d_x * E

    W2 = (jnp.eye(d_x, dtype=jnp.float32)[:, :, None]
          * time_w[None, :, :]).reshape(d_x, TD)
    b_flat = time_b.reshape(1, TD)
    vt_W1 = vt_W[1:]
    w0row = vt_W[0:1]
    crow = (vt_b + given_table[1])[None, :]
    drow = (given_table[0] - given_table[1])[None, :]

    sp_head = _space_sc_head(space_table, bs, L, d_model)
    sp_head = sp_head.reshape(bs, d_y, L, d_model)
    vt = _val_time_tc(y, x, local_emb_table, W2, b_flat, vt_W1, w0row, crow,
                      drow)
    sp = _space_tc_tail(space_table, sp_head, bs, L, d_model)
    return (vt, sp)


# speedup vs baseline: 1.2866x; 1.2866x over previous
import jax
import jax.numpy as jnp
from jax.experimental import pallas as pl
from jax.experimental.pallas import tpu as pltpu

_TB = 256


def _vt_body(y_ref, x_ref, loc_ref, W2_ref, bf_ref, W1_ref, w0_ref, c_ref,
             d_ref, ovt_ref):
    xb = x_ref[0]
    xb = jnp.where(jnp.isnan(xb), 0.0, xb)
    xa = jnp.dot(xb, W2_ref[...], preferred_element_type=jnp.float32)
    xa = xa + bf_ref[...]
    k = jax.lax.broadcasted_iota(jnp.int32, xa.shape, 1) % 6
    feat = jnp.where(k == 0, xa, jnp.sin(xa))
    tp = jnp.dot(feat, W1_ref[...], preferred_element_type=jnp.float32)
    base = loc_ref[...] + tp + c_ref[...]
    yb = y_ref[0]
    nanm = jnp.isnan(yb)
    y0 = jnp.where(nanm, 0.0, yb)
    nf = nanm.astype(jnp.float32)
    w0 = w0_ref[...]
    dl = d_ref[...]
    for i in range(8):
        ovt_ref[0, i] = base + y0[:, i:i + 1] * w0 + nf[:, i:i + 1] * dl


def _sp_body(sp_ref, out_ref):
    i = pl.program_id(1)
    row = sp_ref[pl.ds(i, 1), :]
    out_ref[0, 0] = jnp.broadcast_to(row, out_ref.shape[2:])


def kernel(y, x, local_emb_table, time_w, time_b, vt_W, vt_b, space_table,
           given_table):
    bs, L, d_y = y.shape
    d_x = x.shape[-1]
    d_model = local_emb_table.shape[-1]
    E = time_w.shape[1]
    TD = d_x * E

    W2 = (jnp.eye(d_x, dtype=jnp.float32)[:, :, None]
          * time_w[None, :, :]).reshape(d_x, TD)
    b_flat = time_b.reshape(1, TD)
    vt_W1 = vt_W[1:]
    w0row = vt_W[0:1]
    crow = (vt_b + given_table[1])[None, :]
    drow = (given_table[0] - given_table[1])[None, :]

    nt = L // _TB
    vt4 = pl.pallas_call(
        _vt_body,
        grid=(bs, nt),
        in_specs=[
            pl.BlockSpec((1, _TB, d_y), lambda b, t: (b, t, 0)),
            pl.BlockSpec((1, _TB, d_x), lambda b, t: (b, t, 0)),
            pl.BlockSpec((_TB, d_model), lambda b, t: (t, 0)),
            pl.BlockSpec((d_x, TD), lambda b, t: (0, 0)),
            pl.BlockSpec((1, TD), lambda b, t: (0, 0)),
            pl.BlockSpec((TD, d_model), lambda b, t: (0, 0)),
            pl.BlockSpec((1, d_model), lambda b, t: (0, 0)),
            pl.BlockSpec((1, d_model), lambda b, t: (0, 0)),
            pl.BlockSpec((1, d_model), lambda b, t: (0, 0)),
        ],
        out_specs=pl.BlockSpec((1, d_y, _TB, d_model),
                               lambda b, t: (b, 0, t, 0)),
        out_shape=jax.ShapeDtypeStruct((bs, d_y, L, d_model), jnp.float32),
        compiler_params=pltpu.CompilerParams(
            dimension_semantics=("parallel", "parallel")),
    )(y, x, local_emb_table, W2, b_flat, vt_W1, w0row, crow, drow)

    sp4 = pl.pallas_call(
        _sp_body,
        grid=(bs, d_y),
        in_specs=[pl.BlockSpec((d_y, d_model), lambda b, i: (0, 0))],
        out_specs=pl.BlockSpec((1, 1, L, d_model), lambda b, i: (b, i, 0, 0)),
        out_shape=jax.ShapeDtypeStruct((bs, d_y, L, d_model), jnp.float32),
        compiler_params=pltpu.CompilerParams(
            dimension_semantics=("parallel", "parallel")),
    )(space_table)

    return (vt4.reshape(bs, d_y * L, d_model),
            sp4.reshape(bs, d_y * L, d_model))


# speedup vs baseline: 1.5872x; 1.2336x over previous
import jax
import jax.numpy as jnp
from jax.experimental import pallas as pl
from jax.experimental.pallas import tpu as pltpu

_TB = 256


def _body(yt_ref, xt_ref, loc_ref, twb_ref, vtW_ref, vtb_ref, g_ref, sp_ref,
          ovt_ref, osp_ref):
    d_y = yt_ref.shape[1]
    d_x = xt_ref.shape[1]
    TD = twb_ref.shape[1]
    E = TD // d_x

    xb = jnp.transpose(xt_ref[0])
    xb = jnp.where(jnp.isnan(xb), 0.0, xb)
    sel = (jax.lax.broadcasted_iota(jnp.int32, (d_x, TD), 0)
           == jax.lax.broadcasted_iota(jnp.int32, (d_x, TD), 1) // E)
    W2 = jnp.where(sel, jnp.broadcast_to(twb_ref[0:1, :], (d_x, TD)), 0.0)
    xa = jnp.dot(xb, W2, preferred_element_type=jnp.float32)
    xa = xa + twb_ref[1:2, :]
    k = jax.lax.broadcasted_iota(jnp.int32, xa.shape, 1) % E
    feat = jnp.where(k == 0, xa, jnp.sin(xa))
    tp = jnp.dot(feat, vtW_ref[1:, :], preferred_element_type=jnp.float32)
    crow = vtb_ref[...] + g_ref[1:2, :]
    drow = g_ref[0:1, :] - g_ref[1:2, :]
    base = loc_ref[...] + tp + crow
    yb = jnp.transpose(yt_ref[0])
    nanm = jnp.isnan(yb)
    y0 = jnp.where(nanm, 0.0, yb)
    nf = nanm.astype(jnp.float32)
    w0 = vtW_ref[0:1, :]
    for i in range(d_y):
        ovt_ref[0, i] = base + y0[:, i:i + 1] * w0 + nf[:, i:i + 1] * drow
        osp_ref[0, i] = jnp.broadcast_to(sp_ref[i:i + 1, :], base.shape)


def kernel(y, x, local_emb_table, time_w, time_b, vt_W, vt_b, space_table,
           given_table):
    bs, L, d_y = y.shape
    d_x = x.shape[-1]
    d_model = local_emb_table.shape[-1]
    E = time_w.shape[1]
    TD = d_x * E

    yt = y.transpose(0, 2, 1)
    xt = x.transpose(0, 2, 1)
    twb = jnp.stack([time_w.reshape(TD), time_b.reshape(TD)])
    vtb_row = vt_b.reshape(1, d_model)

    nt = L // _TB
    out4 = [jax.ShapeDtypeStruct((bs, d_y, L, d_model), jnp.float32)] * 2
    vt4, sp4 = pl.pallas_call(
        _body,
        grid=(nt, bs),
        in_specs=[
            pl.BlockSpec((1, d_y, _TB), lambda t, b: (b, 0, t)),
            pl.BlockSpec((1, d_x, _TB), lambda t, b: (b, 0, t)),
            pl.BlockSpec((_TB, d_model), lambda t, b: (t, 0)),
            pl.BlockSpec((2, TD), lambda t, b: (0, 0)),
            pl.BlockSpec((1 + TD, d_model), lambda t, b: (0, 0)),
            pl.BlockSpec((1, d_model), lambda t, b: (0, 0)),
            pl.BlockSpec((2, d_model), lambda t, b: (0, 0)),
            pl.BlockSpec((d_y, d_model), lambda t, b: (0, 0)),
        ],
        out_specs=[
            pl.BlockSpec((1, d_y, _TB, d_model), lambda t, b: (b, 0, t, 0)),
            pl.BlockSpec((1, d_y, _TB, d_model), lambda t, b: (b, 0, t, 0)),
        ],
        out_shape=out4,
        compiler_params=pltpu.CompilerParams(
            dimension_semantics=("parallel", "parallel")),
    )(yt, xt, local_emb_table, twb, vt_W, vtb_row, given_table, space_table)

    return (vt4.reshape(bs, d_y * L, d_model),
            sp4.reshape(bs, d_y * L, d_model))


# speedup vs baseline: 1.6306x; 1.0274x over previous
import jax
import jax.numpy as jnp
from jax.experimental import pallas as pl
from jax.experimental.pallas import tpu as pltpu

_TB = 256


def _body(yt_ref, xt_ref, loc_ref, tw_ref, tb_ref, vtW_ref, vtb_ref, g_ref,
          sp_ref, ovt_ref, osp_ref):
    d_y = yt_ref.shape[1]
    d_x = xt_ref.shape[1]
    E = tw_ref.shape[1]
    TD = d_x * E

    xb = jnp.transpose(xt_ref[0])
    xb = jnp.where(jnp.isnan(xb), 0.0, xb)
    wflat = jnp.concatenate([tw_ref[j:j + 1, :] for j in range(d_x)], axis=1)
    bflat = jnp.concatenate([tb_ref[j:j + 1, :] for j in range(d_x)], axis=1)
    sel = (jax.lax.broadcasted_iota(jnp.int32, (d_x, TD), 0)
           == jax.lax.broadcasted_iota(jnp.int32, (d_x, TD), 1) // E)
    W2 = jnp.where(sel, jnp.broadcast_to(wflat, (d_x, TD)), 0.0)
    xa = jnp.dot(xb, W2, preferred_element_type=jnp.float32)
    xa = xa + bflat
    k = jax.lax.broadcasted_iota(jnp.int32, xa.shape, 1) % E
    feat = jnp.where(k == 0, xa, jnp.sin(xa))
    tp = jnp.dot(feat, vtW_ref[1:, :], preferred_element_type=jnp.float32)
    crow = vtb_ref[...].reshape(1, -1) + g_ref[1:2, :]
    drow = g_ref[0:1, :] - g_ref[1:2, :]
    base = loc_ref[...] + tp + crow
    yb = jnp.transpose(yt_ref[0])
    nanm = jnp.isnan(yb)
    y0 = jnp.where(nanm, 0.0, yb)
    nf = nanm.astype(jnp.float32)
    w0 = vtW_ref[0:1, :]
    for i in range(d_y):
        ovt_ref[0, i] = base + y0[:, i:i + 1] * w0 + nf[:, i:i + 1] * drow
        osp_ref[0, i] = jnp.broadcast_to(sp_ref[i:i + 1, :], base.shape)


def kernel(y, x, local_emb_table, time_w, time_b, vt_W, vt_b, space_table,
           given_table):
    bs, L, d_y = y.shape
    d_x = x.shape[-1]
    d_model = local_emb_table.shape[-1]
    E = time_w.shape[1]
    TD = d_x * E

    yt = y.transpose(0, 2, 1)
    xt = x.transpose(0, 2, 1)

    nt = L // _TB
    out4 = [jax.ShapeDtypeStruct((bs, d_y, L, d_model), jnp.float32)] * 2
    vt4, sp4 = pl.pallas_call(
        _body,
        grid=(nt, bs),
        in_specs=[
            pl.BlockSpec((1, d_y, _TB), lambda t, b: (b, 0, t)),
            pl.BlockSpec((1, d_x, _TB), lambda t, b: (b, 0, t)),
            pl.BlockSpec((_TB, d_model), lambda t, b: (t, 0)),
            pl.BlockSpec((d_x, E), lambda t, b: (0, 0)),
            pl.BlockSpec((d_x, E), lambda t, b: (0, 0)),
            pl.BlockSpec((1 + TD, d_model), lambda t, b: (0, 0)),
            pl.BlockSpec((d_model,), lambda t, b: (0,)),
            pl.BlockSpec((2, d_model), lambda t, b: (0, 0)),
            pl.BlockSpec((d_y, d_model), lambda t, b: (0, 0)),
        ],
        out_specs=[
            pl.BlockSpec((1, d_y, _TB, d_model), lambda t, b: (b, 0, t, 0)),
            pl.BlockSpec((1, d_y, _TB, d_model), lambda t, b: (b, 0, t, 0)),
        ],
        out_shape=out4,
        compiler_params=pltpu.CompilerParams(
            dimension_semantics=("parallel", "parallel")),
    )(yt, xt, local_emb_table, time_w, time_b, vt_W, vt_b, given_table,
      space_table)

    return (vt4.reshape(bs, d_y * L, d_model),
            sp4.reshape(bs, d_y * L, d_model))
